# Initial kernel scaffold; baseline (speedup 1.0000x reference)
import jax
import jax.numpy as jnp
from jax.experimental import pallas as pl


def kernel(logits):
    raise NotImplementedError("write your pallas kernel here")


# speedup vs baseline: 41.2254x; 41.2254x over previous
import functools
import math

import jax
import jax.numpy as jnp
import numpy as np
from jax import lax
from jax.experimental import pallas as pl
from jax.experimental.pallas import tpu as pltpu
from jax.experimental.pallas import tpu_sc as plsc

B, W, T, V = 16, 4, 8, 32000
E = 2
BOS, PAD, EOS = 1, 0, 2
L = 16
NVEC = V // L
ROWS = B * W * T
NEGF = np.float32(-np.inf)
IMAX = np.int32(2**31 - 1)
LN2 = 0.6931471805599453


def _recip(x):
    xb = lax.bitcast_convert_type(x, jnp.int32)
    r = lax.bitcast_convert_type(np.int32(0x7EF311C3) - xb, jnp.float32)
    for _ in range(3):
        r = r * (2.0 - x * r)
    return r


def _ln(s):
    sb = lax.bitcast_convert_type(s, jnp.int32)
    e = (sb >> 23) - 127
    fb = (sb & 0x007FFFFF) | 0x3F800000
    f = lax.bitcast_convert_type(fb, jnp.float32)
    z = (f - 1.0) * _recip(f + 1.0)
    z2 = z * z
    p = 2.0 / 11.0
    for c in (2.0 / 9.0, 2.0 / 7.0, 2.0 / 5.0, 2.0 / 3.0, 2.0):
        p = p * z2 + c
    return e.astype(jnp.float32) * jnp.float32(LN2) + z * p


def _sel4(p, xs):
    return jnp.where(p == 0, xs[0],
                     jnp.where(p == 1, xs[1],
                               jnp.where(p == 2, xs[2], xs[3])))


NCH = 8
CHV = NVEC // NCH


def _tree_max8(vs):
    return jnp.maximum(
        jnp.maximum(jnp.maximum(vs[0], vs[1]), jnp.maximum(vs[2], vs[3])),
        jnp.maximum(jnp.maximum(vs[4], vs[5]), jnp.maximum(vs[6], vs[7])))


def _row_topk(buf, resv, resi, r):
    i16 = lax.iota(jnp.int32, L)

    cmx = []
    for c in range(NCH):
        def l1(i, mx):
            return jnp.maximum(mx, buf[pl.ds(i * L, L)])
        cmx.append(lax.fori_loop(c * CHV, (c + 1) * CHV, l1,
                                 jnp.full((L,), NEGF, jnp.float32)))
    m = jnp.max(_tree_max8(cmx))
    work = list(cmx)
    sigma = m
    for k in range(4):
        sigma = jnp.max(_tree_max8(work))
        if k < 3:
            work = [jnp.where(w == sigma, NEGF, w) for w in work]

    def l23(i, c):
        acc, pool, pooli = c
        v = buf[pl.ds(i * L, L)]
        acc = acc + jnp.exp(v - m)

        def ins(op):
            po, poi = op
            iv = i16 + i * L
            sv, siv = plsc.sort_key_val(v, iv, descending=True)
            gs, gis = plsc.sort_key_val(po, poi, descending=True)
            npo = jnp.where(i16 < 11, gs, lax.rev(sv, (0,)))
            npoi = jnp.where(i16 < 11, gis, lax.rev(siv, (0,)))
            return npo, npoi

        pool, pooli = lax.cond(jnp.any(v >= sigma), ins,
                               lambda op: op, (pool, pooli))
        return acc, pool, pooli

    acc, pool, pooli = lax.fori_loop(
        0, NVEC, l23,
        (jnp.zeros((L,), jnp.float32), jnp.full((L,), NEGF, jnp.float32),
         jnp.zeros((L,), jnp.int32)))
    lse = m + _ln(jnp.sum(acc))

    lp = pool - lse
    rowv = jnp.zeros((L,), jnp.float32)
    rowi = jnp.zeros((L,), jnp.int32)
    for k in range(4):
        mv = jnp.max(lp)
        mi = jnp.min(jnp.where(lp == mv, pooli, IMAX))
        rowv = jnp.where(i16 == k, mv, rowv)
        rowi = jnp.where(i16 == k, mi, rowi)
        lp = jnp.where((lp == mv) & (pooli == mi), NEGF, lp)
    resv[pl.ds(r * 16, 16)] = rowv
    resi[pl.ds(r * 16, 16)] = rowi


def _recursion(candv, candi, penv, seqbuf, scobuf, lenbuf):
    i16 = lax.iota(jnp.int32, L)
    pv = penv[pl.ds(0, 16)]

    def recv(w, t):
        base = (w * 8 + t) * 16
        return candv[pl.ds(base, 16)], candi[pl.ds(base, 16)]

    cv0, ci0 = recv(0, 0)
    seqv, scov, S, last, npd = [], [], [], [], []
    for w in range(W):
        tok = ci0[w]
        v = cv0[w]
        sq = jnp.where(i16 == 0, jnp.int32(BOS), jnp.int32(PAD))
        sq = jnp.where(i16 == 1, tok, sq)
        sc = jnp.where(i16 == 1, v, jnp.float32(0.0))
        seqv.append(sq)
        scov.append(sc)
        S.append(v)
        last.append(tok)
        npd.append(jnp.int32(1) + (tok != PAD).astype(jnp.int32))

    for t in range(1, T):
        cs, cv, ct = [], [], []
        for w in range(W):
            done = (last[w] == PAD) | (last[w] == EOS)
            cvw, ciw = recv(w, t)
            for e in range(E):
                val = jnp.where(done,
                                jnp.float32(0.0) if e == 0 else NEGF,
                                cvw[e])
                tok = jnp.where(done,
                                jnp.int32(PAD) if e == 0 else jnp.int32(1),
                                ciw[e])
                ln = npd[w] + (tok != PAD).astype(jnp.int32)
                ipen = pv[1]
                for n in range(2, t + 3):
                    ipen = jnp.where(ln == n, pv[n], ipen)
                cs.append((S[w] + val) * ipen)
                cv.append(val)
                ct.append(tok)
        nseq, nsco, nS, nlast, nnpd = [], [], [], [], []
        for k in range(4):
            best, bi, bv, bt = cs[0], jnp.int32(0), cv[0], ct[0]
            for i in range(1, 8):
                g = cs[i] > best
                best = jnp.where(g, cs[i], best)
                bi = jnp.where(g, jnp.int32(i), bi)
                bv = jnp.where(g, cv[i], bv)
                bt = jnp.where(g, ct[i], bt)
            p = bi >> 1
            sq = jnp.where(i16 == t + 1, bt, _sel4(p, seqv))
            sc = jnp.where(i16 == t + 1, bv, _sel4(p, scov))
            nseq.append(sq)
            nsco.append(sc)
            nS.append(_sel4(p, S) + bv)
            nlast.append(bt)
            nnpd.append(_sel4(p, npd) + (bt != PAD).astype(jnp.int32))
            cs = [jnp.where(jnp.int32(i) == bi, NEGF, cs[i])
                  for i in range(8)]
        seqv, scov, S, last, npd = nseq, nsco, nS, nlast, nnpd

    lnv = jnp.zeros((L,), jnp.int32)
    for w in range(W):
        seqbuf[pl.ds(w * 16, 16)] = seqv[w]
        scobuf[pl.ds(w * 16, 16)] = scov[w]
        lnv = jnp.where(i16 == w, npd[w], lnv)
    lenbuf[pl.ds(0, 16)] = lnv


def _sc_body(logits_ref, pen_ref, seq_out, sco_out, len_out,
             buf0, buf1, resv, resi, candv, candi, penv,
             seqbuf, scobuf, lenbuf, shv, shi, sem0, sem1):
    c = lax.axis_index("c")
    s = lax.axis_index("s")
    b = c * 8 + (s >> 1)
    half = s & 1
    row_base = b * 32 + half * 16

    bufs = (buf0, buf1)
    sems = (sem0, sem1)
    h = pltpu.async_copy(logits_ref.at[row_base], buf0, sem0)
    for r in range(16):
        h.wait()
        if r < 15:
            h = pltpu.async_copy(logits_ref.at[row_base + r + 1],
                                 bufs[(r + 1) % 2], sems[(r + 1) % 2])
        _row_topk(bufs[r % 2], resv, resi, r)

    pltpu.sync_copy(resv, shv.at[pl.ds(s * 256, 256)])
    pltpu.sync_copy(resi, shi.at[pl.ds(s * 256, 256)])
    plsc.subcore_barrier()

    @pl.when(half == 0)
    def _():
        pltpu.sync_copy(shv.at[pl.ds(s * 256, 512)], candv)
        pltpu.sync_copy(shi.at[pl.ds(s * 256, 512)], candi)
        pltpu.sync_copy(pen_ref, penv)
        _recursion(candv, candi, penv, seqbuf, scobuf, lenbuf)
        pltpu.sync_copy(seqbuf, seq_out.at[b])
        pltpu.sync_copy(scobuf, sco_out.at[b])
        pltpu.sync_copy(lenbuf, len_out.at[b])


@jax.jit
def kernel(logits):
    logits2d = logits.reshape(ROWS, V)
    pen = jnp.float32(1.0) / (
        jnp.power(5.0 + jnp.arange(16, dtype=jnp.float32),
                  jnp.float32(0.6)) / jnp.float32(6.0 ** 0.6))
    kfn = pl.kernel(
        _sc_body,
        out_type=[
            jax.ShapeDtypeStruct((B, 64), jnp.int32),
            jax.ShapeDtypeStruct((B, 64), jnp.float32),
            jax.ShapeDtypeStruct((B, 16), jnp.int32),
        ],
        mesh=plsc.VectorSubcoreMesh(core_axis_name="c", subcore_axis_name="s"),
        compiler_params=pltpu.CompilerParams(needs_layout_passes=False),
        scratch_types=[
            pltpu.VMEM((V,), jnp.float32),
            pltpu.VMEM((V,), jnp.float32),
            pltpu.VMEM((256,), jnp.float32),
            pltpu.VMEM((256,), jnp.int32),
            pltpu.VMEM((512,), jnp.float32),
            pltpu.VMEM((512,), jnp.int32),
            pltpu.VMEM((16,), jnp.float32),
            pltpu.VMEM((64,), jnp.int32),
            pltpu.VMEM((64,), jnp.float32),
            pltpu.VMEM((16,), jnp.int32),
            pltpu.VMEM_SHARED((4096,), jnp.float32),
            pltpu.VMEM_SHARED((4096,), jnp.int32),
            pltpu.SemaphoreType.DMA,
            pltpu.SemaphoreType.DMA,
        ],
    )
    seq_o, sco_o, len_o = kfn(logits2d, pen)
    seq = seq_o.reshape(B, W, 16)[:, :, :9]
    sco = sco_o.reshape(B, W, 16)[:, :, :9]
    return (seq, sco, len_o[:, :4])
